# baseline (speedup 1.0000x reference)
import functools

import jax
import jax.numpy as jnp
from jax import lax
from jax.experimental import pallas as pl
from jax.experimental.pallas import tpu as pltpu
from jax.experimental.pallas import tpu_sc as plsc

B = 1024
ESM_DIM = 1280
EMB_DIM = 1024
N_INDEX = 100000
K = 8
ATT_TAU = 1.0
NUM_CLASSES = 2

BN = 1000
NB = N_INDEX // BN
NEG = float("-inf")
IMAX = 2**31 - 1


def _topk_body(q_ref, e_ref, out_ref, qn_ref, nio_ref, sa_ref, sb_ref,
               val_ref, idx_ref):
    i = pl.program_id(0)

    @pl.when(i == 0)
    def _():
        q = q_ref[...]
        n = jnp.sqrt(jnp.sum(q * q, axis=1, keepdims=True))
        qn_ref[...] = q / jnp.maximum(n, 1e-12)
        nio_ref[...] = -(lax.broadcasted_iota(
            jnp.int32, (BN, B), 0).astype(jnp.float32) + 2048.0)

    for p in (0, 1):
        wbuf, rbuf = (sa_ref, sb_ref) if p == 0 else (sb_ref, sa_ref)

        @pl.when(lax.rem(i, 2) == p)
        def _(wbuf=wbuf, rbuf=rbuf):
            bidx = jnp.maximum(i - 1, 0)
            bnf = (bidx * BN).astype(jnp.float32)
            nio = nio_ref[...]
            v = rbuf[...]
            qn = qn_ref[...]
            mm_chunks = ((0, 256), (256, 256), (512, 256), (768, 232))
            for k in range(K):
                if k < len(mm_chunks):
                    lo, sz = mm_chunks[k]
                    wbuf[lo:lo + sz, :] = lax.dot_general(
                        e_ref[lo:lo + sz, :], qn,
                        (((1,), (1,)), ((), ())),
                        preferred_element_type=jnp.float32)
                m = jnp.max(v, axis=0)
                s = jnp.where(v == m[None, :], nio, 0.0)
                a = jnp.min(s, axis=0)
                val_ref[bidx, k] = m
                idx_ref[bidx, k] = a + (2048.0 - bnf)
                v = v + s * 1e35

    @pl.when(i == NB)
    def _():
        v = val_ref[...]
        ids = idx_ref[...]
        for k in range(K):
            m = jnp.max(jnp.max(v, axis=0), axis=0)
            e = v == m[None, None, :]
            a = jnp.max(jnp.max(jnp.where(e, ids, NEG), axis=0), axis=0)
            out_ref[k] = (-a).astype(jnp.int32)
            v = jnp.where(e, NEG, v)


def _topk_indices(query_esm_pooled, index_emb):
    return pl.pallas_call(
        _topk_body,
        grid=(NB + 1,),
        in_specs=[
            pl.BlockSpec((B, ESM_DIM), lambda i: (0, 0)),
            pl.BlockSpec((BN, ESM_DIM), lambda i: (jnp.minimum(i, NB - 1), 0)),
        ],
        out_specs=pl.BlockSpec((K, B), lambda i: (0, 0)),
        out_shape=jax.ShapeDtypeStruct((K, B), jnp.int32),
        scratch_shapes=[
            pltpu.VMEM((B, ESM_DIM), jnp.float32),
            pltpu.VMEM((BN, B), jnp.float32),
            pltpu.VMEM((BN, B), jnp.float32),
            pltpu.VMEM((BN, B), jnp.float32),
            pltpu.VMEM((NB, K, B), jnp.float32),
            pltpu.VMEM((NB, K, B), jnp.float32),
        ],
    )(query_esm_pooled, index_emb)


_NC, _NS = 2, 16
_NW = _NC * _NS
_RPW = (B * K) // _NW
_CH = 64
_NCH = _RPW // _CH


def _sc_gather(idx_flat, index_emb, index_labels):
    mesh = plsc.VectorSubcoreMesh(core_axis_name="c", subcore_axis_name="s")

    @functools.partial(
        pl.kernel,
        mesh=mesh,
        out_type=[
            jax.ShapeDtypeStruct((B * K, ESM_DIM), jnp.float32),
            jax.ShapeDtypeStruct((B * K,), jnp.int32),
        ],
        scratch_types=[
            pltpu.VMEM((_NCH, _CH), jnp.int32),
            pltpu.VMEM((_RPW,), jnp.int32),
            pltpu.VMEM((_CH, ESM_DIM), jnp.float32),
            pltpu.VMEM((_RPW,), jnp.int32),
            pltpu.SemaphoreType.DMA,
        ],
    )
    def gather_k(idx_hbm, table_hbm, labels_hbm, out_rows, out_lbls,
                 idx_v, lidx_v, rows_v, lbl_v, sem):
        wid = lax.axis_index("s") * _NC + lax.axis_index("c")
        base = wid * _RPW
        pltpu.sync_copy(idx_hbm.at[pl.ds(base, _RPW)], lidx_v)
        pltpu.async_copy(labels_hbm.at[lidx_v], lbl_v, sem).wait()
        pltpu.sync_copy(lbl_v, out_lbls.at[pl.ds(base, _RPW)])
        for c in range(_NCH):
            pltpu.sync_copy(idx_hbm.at[pl.ds(base + c * _CH, _CH)],
                            idx_v.at[c])
            pltpu.async_copy(table_hbm.at[idx_v.at[c]], rows_v, sem).wait()
            pltpu.sync_copy(rows_v,
                            out_rows.at[pl.ds(base + c * _CH, _CH)])

    return gather_k(idx_flat, index_emb, index_labels)


BB = 256
NBB = B // BB
_SCALE = (EMB_DIM ** 0.5) * ATT_TAU
_EPS = 1e-6


def _fuse_body(me_ref, ng_ref, lbl_ref, wp_ref, bp_ref, wc_ref, bc_ref,
               logits_ref, loss_ref, acc_ref):
    j = pl.program_id(0)

    me = me_ref[...]
    ng = ng_ref[...].reshape(K * BB, ESM_DIM)
    proj = lax.dot_general(ng, wp_ref[...], (((1,), (1,)), ((), ())),
                           preferred_element_type=jnp.float32)
    proj = proj + bp_ref[...]
    proj3 = proj.reshape(K, BB, EMB_DIM)

    s = jnp.sum(proj3 * me[None, :, :], axis=2) / _SCALE
    smax = jnp.max(s, axis=0)
    e = jnp.exp(s - smax[None, :])
    att = e / jnp.sum(e, axis=0)[None, :]
    z = jnp.sum(proj3 * att[:, :, None], axis=0)

    wc = wc_ref[...]
    logits = (
        lax.dot_general(me, wc[:, :EMB_DIM], (((1,), (1,)), ((), ())),
                        preferred_element_type=jnp.float32)
        + lax.dot_general(z, wc[:, EMB_DIM:], (((1,), (1,)), ((), ())),
                          preferred_element_type=jnp.float32)
        + bc_ref[...]
    )
    logits_ref[...] = logits

    lbl = lbl_ref[...]
    mp = (lbl == 1).astype(jnp.float32)
    mn = (lbl == 0).astype(jnp.float32)
    pos = (jnp.sum(proj3 * mp[:, :, None], axis=0)
           / (jnp.sum(mp, axis=0)[:, None] + _EPS))
    neg = (jnp.sum(proj3 * mn[:, :, None], axis=0)
           / (jnp.sum(mn, axis=0)[:, None] + _EPS))
    sA = jnp.sum((me - pos) ** 2)
    sB = jnp.sum((me - neg) ** 2)
    accA = jnp.where(j == 0, 0.0, acc_ref[0]) + sA
    accB = jnp.where(j == 0, 0.0, acc_ref[1]) + sB
    acc_ref[0] = accA
    acc_ref[1] = accB
    denom = jnp.float32(B * EMB_DIM)
    loss_ref[0, 0] = accA / denom - 0.5 * (accB / denom)


def _fuse(model_embs, neigh, neigh_lbl, W_proj, b_proj, W_clf, b_clf):
    ng3 = neigh.reshape(K, B, ESM_DIM)
    lbl2 = neigh_lbl.reshape(K, B)
    logits, loss = pl.pallas_call(
        _fuse_body,
        grid=(NBB,),
        in_specs=[
            pl.BlockSpec((BB, EMB_DIM), lambda j: (j, 0)),
            pl.BlockSpec((K, BB, ESM_DIM), lambda j: (0, j, 0)),
            pl.BlockSpec((K, BB), lambda j: (0, j)),
            pl.BlockSpec((EMB_DIM, ESM_DIM), lambda j: (0, 0)),
            pl.BlockSpec((1, EMB_DIM), lambda j: (0, 0)),
            pl.BlockSpec((NUM_CLASSES, 2 * EMB_DIM), lambda j: (0, 0)),
            pl.BlockSpec((1, NUM_CLASSES), lambda j: (0, 0)),
        ],
        out_specs=[
            pl.BlockSpec((BB, NUM_CLASSES), lambda j: (j, 0)),
            pl.BlockSpec((1, 1), lambda j: (0, 0),
                         memory_space=pltpu.SMEM),
        ],
        out_shape=[
            jax.ShapeDtypeStruct((B, NUM_CLASSES), jnp.float32),
            jax.ShapeDtypeStruct((1, 1), jnp.float32),
        ],
        scratch_shapes=[pltpu.SMEM((2,), jnp.float32)],
    )(model_embs, ng3, lbl2, W_proj, b_proj.reshape(1, EMB_DIM),
      W_clf, b_clf.reshape(1, NUM_CLASSES))
    return logits, loss[0, 0]


def kernel(query_seq_batch, query_esm_pooled, model_embs, index_emb,
           index_labels, W_proj, b_proj, W_clf, b_clf):
    del query_seq_batch
    topk_t = _topk_indices(query_esm_pooled, index_emb)
    idx_flat = topk_t.reshape(B * K)
    neigh, neigh_lbl = _sc_gather(idx_flat, index_emb, index_labels)
    return _fuse(model_embs, neigh, neigh_lbl, W_proj, b_proj, W_clf, b_clf)


# speedup vs baseline: 1.2806x; 1.2806x over previous
import functools

import jax
import jax.numpy as jnp
from jax import lax
from jax.experimental import pallas as pl
from jax.experimental.pallas import tpu as pltpu
from jax.experimental.pallas import tpu_sc as plsc

B = 1024
ESM_DIM = 1280
EMB_DIM = 1024
N_INDEX = 100000
K = 8
ATT_TAU = 1.0
NUM_CLASSES = 2

BN = 1000
NB = N_INDEX // BN
NEG = float("-inf")
IMAX = 2**31 - 1


def _topk_body(q_ref, e_ref, out_ref, qn_ref, nio_ref, sa_ref, sb_ref,
               val_ref, idx_ref):
    i = pl.program_id(0)

    @pl.when(i == 0)
    def _():
        q = q_ref[...]
        n = jnp.sqrt(jnp.sum(q * q, axis=1, keepdims=True))
        qn_ref[...] = q / jnp.maximum(n, 1e-12)
        nio_ref[...] = -lax.broadcasted_iota(
            jnp.int32, (BN, B), 0).astype(jnp.float32)

    for p in (0, 1):
        wbuf, rbuf = (sa_ref, sb_ref) if p == 0 else (sb_ref, sa_ref)

        @pl.when(lax.rem(i, 2) == p)
        def _(wbuf=wbuf, rbuf=rbuf):
            bidx = jnp.maximum(i - 1, 0)
            bnf = (bidx * BN).astype(jnp.float32)
            nio = nio_ref[...]
            v = rbuf[...]
            qn = qn_ref[...]
            mm_chunks = ((0, 256), (256, 256), (512, 256), (768, 232))
            for k in range(K):
                if k < len(mm_chunks):
                    lo, sz = mm_chunks[k]
                    wbuf[lo:lo + sz, :] = lax.dot_general(
                        e_ref[lo:lo + sz, :], qn,
                        (((1,), (1,)), ((), ())),
                        preferred_element_type=jnp.float32)
                m = jnp.max(v, axis=0)
                e = v == m[None, :]
                a = jnp.max(jnp.where(e, nio, NEG), axis=0)
                val_ref[bidx, k] = m
                idx_ref[bidx, k] = a - bnf
                v = jnp.where(e, NEG, v)

    @pl.when(i == NB)
    def _():
        v = val_ref[...]
        ids = idx_ref[...]
        for k in range(K):
            m = jnp.max(jnp.max(v, axis=0), axis=0)
            e = v == m[None, None, :]
            a = jnp.max(jnp.max(jnp.where(e, ids, NEG), axis=0), axis=0)
            out_ref[k] = (-a).astype(jnp.int32)
            v = jnp.where(e, NEG, v)


def _topk_indices(query_esm_pooled, index_emb):
    return pl.pallas_call(
        _topk_body,
        grid=(NB + 1,),
        in_specs=[
            pl.BlockSpec((B, ESM_DIM), lambda i: (0, 0)),
            pl.BlockSpec((BN, ESM_DIM), lambda i: (jnp.minimum(i, NB - 1), 0)),
        ],
        out_specs=pl.BlockSpec((K, B), lambda i: (0, 0)),
        out_shape=jax.ShapeDtypeStruct((K, B), jnp.int32),
        scratch_shapes=[
            pltpu.VMEM((B, ESM_DIM), jnp.float32),
            pltpu.VMEM((BN, B), jnp.float32),
            pltpu.VMEM((BN, B), jnp.float32),
            pltpu.VMEM((BN, B), jnp.float32),
            pltpu.VMEM((NB, K, B), jnp.float32),
            pltpu.VMEM((NB, K, B), jnp.float32),
        ],
    )(query_esm_pooled, index_emb)


_NC, _NS = 2, 16
_NW = _NC * _NS
_RPW = (B * K) // _NW
_CH = 64
_NCH = _RPW // _CH


def _sc_gather(idx_flat, index_emb, index_labels):
    mesh = plsc.VectorSubcoreMesh(core_axis_name="c", subcore_axis_name="s")

    @functools.partial(
        pl.kernel,
        mesh=mesh,
        out_type=[
            jax.ShapeDtypeStruct((B * K, ESM_DIM), jnp.float32),
            jax.ShapeDtypeStruct((B * K,), jnp.int32),
        ],
        scratch_types=[
            pltpu.VMEM((_NCH, _CH), jnp.int32),
            pltpu.VMEM((_RPW,), jnp.int32),
            pltpu.VMEM((_CH, ESM_DIM), jnp.float32),
            pltpu.VMEM((_RPW,), jnp.int32),
            pltpu.SemaphoreType.DMA,
        ],
    )
    def gather_k(idx_hbm, table_hbm, labels_hbm, out_rows, out_lbls,
                 idx_v, lidx_v, rows_v, lbl_v, sem):
        wid = lax.axis_index("s") * _NC + lax.axis_index("c")
        base = wid * _RPW
        pltpu.sync_copy(idx_hbm.at[pl.ds(base, _RPW)], lidx_v)
        pltpu.async_copy(labels_hbm.at[lidx_v], lbl_v, sem).wait()
        pltpu.sync_copy(lbl_v, out_lbls.at[pl.ds(base, _RPW)])
        for c in range(_NCH):
            pltpu.sync_copy(idx_hbm.at[pl.ds(base + c * _CH, _CH)],
                            idx_v.at[c])
            pltpu.async_copy(table_hbm.at[idx_v.at[c]], rows_v, sem).wait()
            pltpu.sync_copy(rows_v,
                            out_rows.at[pl.ds(base + c * _CH, _CH)])

    return gather_k(idx_flat, index_emb, index_labels)


BB = 256
NBB = B // BB
_SCALE = (EMB_DIM ** 0.5) * ATT_TAU
_EPS = 1e-6


def _fuse_body(me_ref, ng_ref, lbl_ref, wp_ref, bp_ref, wc_ref, bc_ref,
               logits_ref, loss_ref, acc_ref):
    j = pl.program_id(0)

    me = me_ref[...]
    ng = ng_ref[...].reshape(K * BB, ESM_DIM)
    proj = lax.dot_general(ng, wp_ref[...], (((1,), (1,)), ((), ())),
                           preferred_element_type=jnp.float32)
    proj = proj + bp_ref[...]
    proj3 = proj.reshape(K, BB, EMB_DIM)

    s = jnp.sum(proj3 * me[None, :, :], axis=2) / _SCALE
    smax = jnp.max(s, axis=0)
    e = jnp.exp(s - smax[None, :])
    att = e / jnp.sum(e, axis=0)[None, :]
    z = jnp.sum(proj3 * att[:, :, None], axis=0)

    wc = wc_ref[...]
    logits = (
        lax.dot_general(me, wc[:, :EMB_DIM], (((1,), (1,)), ((), ())),
                        preferred_element_type=jnp.float32)
        + lax.dot_general(z, wc[:, EMB_DIM:], (((1,), (1,)), ((), ())),
                          preferred_element_type=jnp.float32)
        + bc_ref[...]
    )
    logits_ref[...] = logits

    lbl = lbl_ref[...]
    mp = (lbl == 1).astype(jnp.float32)
    mn = (lbl == 0).astype(jnp.float32)
    pos = (jnp.sum(proj3 * mp[:, :, None], axis=0)
           / (jnp.sum(mp, axis=0)[:, None] + _EPS))
    neg = (jnp.sum(proj3 * mn[:, :, None], axis=0)
           / (jnp.sum(mn, axis=0)[:, None] + _EPS))
    sA = jnp.sum((me - pos) ** 2)
    sB = jnp.sum((me - neg) ** 2)
    accA = jnp.where(j == 0, 0.0, acc_ref[0]) + sA
    accB = jnp.where(j == 0, 0.0, acc_ref[1]) + sB
    acc_ref[0] = accA
    acc_ref[1] = accB
    denom = jnp.float32(B * EMB_DIM)
    loss_ref[0, 0] = accA / denom - 0.5 * (accB / denom)


def _fuse(model_embs, neigh, neigh_lbl, W_proj, b_proj, W_clf, b_clf):
    ng3 = neigh.reshape(K, B, ESM_DIM)
    lbl2 = neigh_lbl.reshape(K, B)
    logits, loss = pl.pallas_call(
        _fuse_body,
        grid=(NBB,),
        in_specs=[
            pl.BlockSpec((BB, EMB_DIM), lambda j: (j, 0)),
            pl.BlockSpec((K, BB, ESM_DIM), lambda j: (0, j, 0)),
            pl.BlockSpec((K, BB), lambda j: (0, j)),
            pl.BlockSpec((EMB_DIM, ESM_DIM), lambda j: (0, 0)),
            pl.BlockSpec((1, EMB_DIM), lambda j: (0, 0)),
            pl.BlockSpec((NUM_CLASSES, 2 * EMB_DIM), lambda j: (0, 0)),
            pl.BlockSpec((1, NUM_CLASSES), lambda j: (0, 0)),
        ],
        out_specs=[
            pl.BlockSpec((BB, NUM_CLASSES), lambda j: (j, 0)),
            pl.BlockSpec((1, 1), lambda j: (0, 0),
                         memory_space=pltpu.SMEM),
        ],
        out_shape=[
            jax.ShapeDtypeStruct((B, NUM_CLASSES), jnp.float32),
            jax.ShapeDtypeStruct((1, 1), jnp.float32),
        ],
        scratch_shapes=[pltpu.SMEM((2,), jnp.float32)],
    )(model_embs, ng3, lbl2, W_proj, b_proj.reshape(1, EMB_DIM),
      W_clf, b_clf.reshape(1, NUM_CLASSES))
    return logits, loss[0, 0]


def kernel(query_seq_batch, query_esm_pooled, model_embs, index_emb,
           index_labels, W_proj, b_proj, W_clf, b_clf):
    del query_seq_batch
    topk_t = _topk_indices(query_esm_pooled, index_emb)
    idx_flat = topk_t.reshape(B * K)
    neigh, neigh_lbl = _sc_gather(idx_flat, index_emb, index_labels)
    return _fuse(model_embs, neigh, neigh_lbl, W_proj, b_proj, W_clf, b_clf)
